# baseline (speedup 1.0000x reference)
import functools

import jax
import jax.numpy as jnp
from jax import lax
from jax.experimental import pallas as pl
from jax.experimental.pallas import tpu as pltpu
from jax.experimental.pallas import tpu_sc as plsc

R = 128
V = 100000
NC = 2
NS = 16
NG = R // 8
HALF_W = 49920
CC = 4992
NCH = HALF_W // CC
SHARED0 = 2 * HALF_W
TAIL0 = SHARED0 + 128
NEG_INF = float("-inf")

_mesh = plsc.VectorSubcoreMesh(core_axis_name="c", subcore_axis_name="s")


@functools.partial(
    pl.kernel,
    out_type=[jax.ShapeDtypeStruct((NC * NS * 16,), jnp.int32),
              jax.ShapeDtypeStruct((NC * NS * 16,), jnp.float32)],
    mesh=_mesh,
    scratch_types=[
        pltpu.VMEM((2, 8, CC), jnp.float32),
        pltpu.VMEM((8, 128), jnp.float32),
        pltpu.VMEM((8, 128), jnp.float32),
        pltpu.VMEM((16,), jnp.float32),
        pltpu.VMEM((16,), jnp.int32),
        pltpu.SemaphoreType.DMA,
        pltpu.SemaphoreType.DMA,
        pltpu.SemaphoreType.DMA,
    ],
)
def _sc_scan(x_hbm, xtail_hbm, outi_hbm, outv_hbm, buf, sbuf, tbuf, sv, si,
             sem0, sem1, semt):
    cid = lax.axis_index("c")
    sid = lax.axis_index("s")
    odd = sid % 2
    g = cid * (NS // 2) + sid // 2
    wid = g * 2 + odd
    row0 = pl.multiple_of(g * 8, 8)
    col0 = pl.multiple_of(odd * HALF_W, 128)
    sems = (sem0, sem1)
    lanes = lax.iota(jnp.int32, 16)
    zero_i = lanes * 0
    neginf_f = zero_i.astype(jnp.float32) + NEG_INF

    def start(k):
        col = pl.multiple_of(col0 + k * CC, 128)
        return pltpu.async_copy(
            x_hbm.at[pl.ds(row0, 8), pl.ds(col, CC)], buf.at[k % 2],
            sems[k % 2])

    def scan_rows(bref, ncols, cbase, bvs, bis):
        for j in range(8):
            ci = lanes + cbase

            def it(i, carry):
                bv, bi, ci = carry
                v = bref[j, pl.ds(i * 16, 16)]
                gt = v > bv
                bv = jnp.maximum(bv, v)
                bi = jnp.where(gt, ci, bi)
                ci = ci + 16
                return bv, bi, ci

            bvs[j], bis[j], _ = lax.fori_loop(
                0, ncols // 16, it, (bvs[j], bis[j], ci), unroll=8)

    bvs = [neginf_f for _ in range(8)]
    bis = [zero_i for _ in range(8)]

    descs = [None, None]
    descs[0] = start(0)
    for k in range(NCH):
        if k + 1 < NCH:
            descs[(k + 1) % 2] = start(k + 1)
        descs[k % 2].wait()
        scan_rows(buf.at[k % 2], CC, col0 + k * CC, bvs, bis)

    shcol = pl.multiple_of(SHARED0, 128)
    pltpu.async_copy(x_hbm.at[pl.ds(row0, 8), pl.ds(shcol, 128)], sbuf,
                     semt).wait()
    scan_rows(sbuf, 128, SHARED0, bvs, bis)
    pltpu.async_copy(xtail_hbm.at[pl.ds(row0, 8), :], tbuf, semt).wait()
    scan_rows(tbuf, 128, TAIL0, bvs, bis)

    _dn = lax.GatherDimensionNumbers(
        offset_dims=(), collapsed_slice_dims=(0,), start_index_map=(0,))

    def perm(x, idx):
        return lax.gather(x, idx[:, None], _dn, (1,),
                          mode=lax.GatherScatterMode.PROMISE_IN_BOUNDS)

    resv = neginf_f
    resi = zero_i
    for j in range(8):
        v, i = bvs[j], bis[j]
        for d in (8, 4, 2, 1):
            pidx = lanes ^ d
            pv_ = perm(v, pidx)
            pi_ = perm(i, pidx)
            gt = pv_ > v
            eq = pv_ == v
            v = jnp.maximum(v, pv_)
            i = jnp.where(gt, pi_, i)
            i = jnp.where(eq, jnp.minimum(i, pi_), i)
        resv = jnp.where(lanes == j, v, resv)
        resi = jnp.where(lanes == j, i, resi)

    si[...] = resi
    pltpu.sync_copy(si, outi_hbm.at[pl.ds(wid * 16, 16)])
    sv[...] = resv
    pltpu.sync_copy(sv, outv_hbm.at[pl.ds(wid * 16, 16)])


@functools.partial(
    pl.kernel,
    out_type=jax.ShapeDtypeStruct((NG * 16,), jnp.int32),
    mesh=_mesh,
    scratch_types=[
        pltpu.VMEM((16,), jnp.float32),
        pltpu.VMEM((16,), jnp.float32),
        pltpu.VMEM((16,), jnp.int32),
        pltpu.VMEM((16,), jnp.int32),
    ],
)
def _sc_merge(pi_hbm, pv_hbm, out_hbm, lvr, hvr, lir, hir):
    cid = lax.axis_index("c")
    sid = lax.axis_index("s")
    odd = sid % 2
    g = cid * (NS // 2) + sid // 2

    @pl.when(odd == 0)
    def _():
        pltpu.sync_copy(pv_hbm.at[pl.ds(g * 32, 16)], lvr)
        pltpu.sync_copy(pv_hbm.at[pl.ds(g * 32 + 16, 16)], hvr)
        pltpu.sync_copy(pi_hbm.at[pl.ds(g * 32, 16)], lir)
        pltpu.sync_copy(pi_hbm.at[pl.ds(g * 32 + 16, 16)], hir)
        lv = lvr[...]
        hv = hvr[...]
        li = lir[...]
        hi = hir[...]
        win = jnp.where(hv > lv, hi, li)
        win = jnp.where(hv == lv, jnp.minimum(hi, li), win)
        lir[...] = win
        pltpu.sync_copy(lir, out_hbm.at[pl.ds(g * 16, 16)])


def kernel(m_logits):
    xtail = jnp.pad(m_logits[:, TAIL0:], ((0, 0), (0, 128 - (V - TAIL0))),
                    constant_values=NEG_INF)
    pi, pv = _sc_scan(m_logits, xtail)
    out = _sc_merge(pi, pv)
    return out.reshape(NG, 16)[:, :8].reshape(R, 1).astype(jnp.int64)


# speedup vs baseline: 1.7944x; 1.7944x over previous
import functools

import jax
import jax.numpy as jnp
from jax import lax
from jax.experimental import pallas as pl
from jax.experimental.pallas import tpu as pltpu
from jax.experimental.pallas import tpu_sc as plsc

B = 128
V = 100000
NC = 2
NS = 16
NW = NC * NS
S = 3136
VC = 448
NCHK = S // VC
NEG_INF = float("-inf")

_mesh = plsc.VectorSubcoreMesh(core_axis_name="c", subcore_axis_name="s")


@functools.partial(
    pl.kernel,
    out_type=[jax.ShapeDtypeStruct((NW * B,), jnp.int32),
              jax.ShapeDtypeStruct((NW * B,), jnp.float32)],
    mesh=_mesh,
    scratch_types=[
        pltpu.VMEM((2, VC, B), jnp.float32),
        pltpu.VMEM((16,), jnp.float32),
        pltpu.VMEM((16,), jnp.int32),
        pltpu.SemaphoreType.DMA,
        pltpu.SemaphoreType.DMA,
    ],
)
def _sc_scan(xt_hbm, outi_hbm, outv_hbm, buf, sv, si, sem0, sem1):
    cid = lax.axis_index("c")
    sid = lax.axis_index("s")
    wid = cid * NS + sid
    v0 = pl.multiple_of((wid * (V - S) // (NW - 1)) // 8 * 8, 8)
    sems = (sem0, sem1)
    lanes = lax.iota(jnp.int32, 16)
    zero_i = lanes * 0
    neginf_f = zero_i.astype(jnp.float32) + NEG_INF

    def start(k):
        return pltpu.async_copy(
            xt_hbm.at[pl.ds(v0 + k * VC, VC), :], buf.at[k % 2],
            sems[k % 2])

    bvs = [neginf_f for _ in range(8)]
    bis = [zero_i for _ in range(8)]

    descs = [None, None]
    descs[0] = start(0)
    for k in range(NCHK):
        if k + 1 < NCHK:
            descs[(k + 1) % 2] = start(k + 1)
        descs[k % 2].wait()
        bref = buf.at[k % 2]
        cbase = v0 + k * VC
        for lg in range(8):

            def it(v, carry):
                bv, bi, civ = carry
                x = bref[v, pl.ds(lg * 16, 16)]
                gt = x > bv
                bv = jnp.maximum(bv, x)
                bi = jnp.where(gt, civ, bi)
                civ = civ + 1
                return bv, bi, civ

            civ0 = zero_i + cbase
            bvs[lg], bis[lg], _ = lax.fori_loop(
                0, VC, it, (bvs[lg], bis[lg], civ0), unroll=8)

    for lg in range(8):
        si[...] = bis[lg]
        pltpu.sync_copy(si, outi_hbm.at[pl.ds(wid * B + lg * 16, 16)])
        sv[...] = bvs[lg]
        pltpu.sync_copy(sv, outv_hbm.at[pl.ds(wid * B + lg * 16, 16)])


@functools.partial(
    pl.kernel,
    out_type=jax.ShapeDtypeStruct((B,), jnp.int32),
    mesh=_mesh,
    scratch_types=[
        pltpu.VMEM((NW * B,), jnp.float32),
        pltpu.VMEM((NW * B,), jnp.int32),
        pltpu.VMEM((16,), jnp.int32),
        pltpu.SemaphoreType.DMA,
        pltpu.SemaphoreType.DMA,
    ],
)
def _sc_merge(pi_hbm, pv_hbm, out_hbm, vbuf, ibuf, si, sem0, sem1):
    cid = lax.axis_index("c")
    sid = lax.axis_index("s")
    lanes = lax.iota(jnp.int32, 16)
    zero_i = lanes * 0
    neginf_f = zero_i.astype(jnp.float32) + NEG_INF

    @pl.when(sid % 4 == 0)
    def _():
        a = cid * 4 + sid // 4
        b0 = a * 16
        d0 = pltpu.async_copy(pv_hbm, vbuf, sem0)
        d1 = pltpu.async_copy(pi_hbm, ibuf, sem1)
        d0.wait()
        d1.wait()
        bv = neginf_f
        bi = zero_i
        for w in range(NW):
            v = vbuf[pl.ds(w * B + b0, 16)]
            i = ibuf[pl.ds(w * B + b0, 16)]
            gt = v > bv
            eq = v == bv
            bv = jnp.maximum(bv, v)
            bi = jnp.where(gt, i, bi)
            bi = jnp.where(eq, jnp.minimum(bi, i), bi)
        si[...] = bi
        pltpu.sync_copy(si, out_hbm.at[pl.ds(b0, 16)])


def kernel(m_logits):
    xt = m_logits.T
    pi, pv = _sc_scan(xt)
    out = _sc_merge(pi, pv)
    return out.reshape(B, 1).astype(jnp.int64)


# speedup vs baseline: 1.9954x; 1.1120x over previous
import functools

import jax
import jax.numpy as jnp
from jax import lax
from jax.experimental import pallas as pl
from jax.experimental.pallas import tpu as pltpu
from jax.experimental.pallas import tpu_sc as plsc

B = 128
V = 100000
NC = 2
NS = 16
NW = NC * NS
S = 3136
VC = 448
NCHK = S // VC
NEG_INF = float("-inf")

_mesh = plsc.VectorSubcoreMesh(core_axis_name="c", subcore_axis_name="s")


@functools.partial(
    pl.kernel,
    out_type=[jax.ShapeDtypeStruct((NW * B,), jnp.int32),
              jax.ShapeDtypeStruct((NW * B,), jnp.float32)],
    mesh=_mesh,
    scratch_types=[
        pltpu.VMEM((2, VC, B), jnp.float32),
        pltpu.VMEM((16,), jnp.float32),
        pltpu.VMEM((16,), jnp.int32),
        pltpu.SemaphoreType.DMA,
        pltpu.SemaphoreType.DMA,
    ],
)
def _sc_scan(xt_hbm, outi_hbm, outv_hbm, buf, sv, si, sem0, sem1):
    cid = lax.axis_index("c")
    sid = lax.axis_index("s")
    wid = cid * NS + sid
    v0 = pl.multiple_of((wid * (V - S) // (NW - 1)) // 8 * 8, 8)
    sems = (sem0, sem1)
    lanes = lax.iota(jnp.int32, 16)
    zero_i = lanes * 0
    neginf_f = zero_i.astype(jnp.float32) + NEG_INF

    def start(k):
        return pltpu.async_copy(
            xt_hbm.at[pl.ds(v0 + k * VC, VC), :], buf.at[k % 2],
            sems[k % 2])

    bvs = [neginf_f for _ in range(8)]
    bis = [zero_i for _ in range(8)]

    descs = [None, None]
    descs[0] = start(0)
    for k in range(NCHK):
        if k + 1 < NCHK:
            descs[(k + 1) % 2] = start(k + 1)
        descs[k % 2].wait()
        bref = buf.at[k % 2]
        cbase = v0 + k * VC

        def it(v, carry):
            accs, civ = carry
            out = []
            for lg in range(8):
                bv, bi = accs[lg]
                x = bref[v, pl.ds(lg * 16, 16)]
                gt = x > bv
                bv = jnp.maximum(bv, x)
                bi = jnp.where(gt, civ, bi)
                out.append((bv, bi))
            return tuple(out), civ + 1

        civ0 = zero_i + cbase
        accs, _ = lax.fori_loop(
            0, VC, it,
            (tuple((bvs[lg], bis[lg]) for lg in range(8)), civ0),
            unroll=4)
        for lg in range(8):
            bvs[lg], bis[lg] = accs[lg]

    for lg in range(8):
        si[...] = bis[lg]
        pltpu.sync_copy(si, outi_hbm.at[pl.ds(wid * B + lg * 16, 16)])
        sv[...] = bvs[lg]
        pltpu.sync_copy(sv, outv_hbm.at[pl.ds(wid * B + lg * 16, 16)])


@functools.partial(
    pl.kernel,
    out_type=jax.ShapeDtypeStruct((B,), jnp.int32),
    mesh=_mesh,
    scratch_types=[
        pltpu.VMEM((NW * B,), jnp.float32),
        pltpu.VMEM((NW * B,), jnp.int32),
        pltpu.VMEM((16,), jnp.int32),
        pltpu.SemaphoreType.DMA,
        pltpu.SemaphoreType.DMA,
    ],
)
def _sc_merge(pi_hbm, pv_hbm, out_hbm, vbuf, ibuf, si, sem0, sem1):
    cid = lax.axis_index("c")
    sid = lax.axis_index("s")
    lanes = lax.iota(jnp.int32, 16)
    zero_i = lanes * 0
    neginf_f = zero_i.astype(jnp.float32) + NEG_INF

    @pl.when(sid % 4 == 0)
    def _():
        a = cid * 4 + sid // 4
        b0 = a * 16
        d0 = pltpu.async_copy(pv_hbm, vbuf, sem0)
        d1 = pltpu.async_copy(pi_hbm, ibuf, sem1)
        d0.wait()
        d1.wait()
        bv = neginf_f
        bi = zero_i
        for w in range(NW):
            v = vbuf[pl.ds(w * B + b0, 16)]
            i = ibuf[pl.ds(w * B + b0, 16)]
            gt = v > bv
            eq = v == bv
            bv = jnp.maximum(bv, v)
            bi = jnp.where(gt, i, bi)
            bi = jnp.where(eq, jnp.minimum(bi, i), bi)
        si[...] = bi
        pltpu.sync_copy(si, out_hbm.at[pl.ds(b0, 16)])


def kernel(m_logits):
    xt = m_logits.T
    pi, pv = _sc_scan(xt)
    out = _sc_merge(pi, pv)
    return out.reshape(B, 1).astype(jnp.int64)
